# baseline (speedup 1.0000x reference)
import jax
import jax.numpy as jnp
from jax import lax
from jax.experimental import pallas as pl
from jax.experimental.pallas import tpu as pltpu


_MIX = 0x45D9F3B
_NB = 999999

_ROWS = 200
_COLS = 16384
_R = 8
_C = _ROWS // _R
_D = 25


def _bucket(v):
    u = v.astype(jnp.uint32)
    h = u ^ (u >> 16)
    h = h * jnp.uint32(_MIX)
    h = h ^ (h >> 16)
    h = h * jnp.uint32(_MIX)
    h = h ^ (h >> 16)
    q = h // jnp.uint32(_NB)
    t = (h - q * jnp.uint32(_NB)).astype(jnp.int32)
    return jnp.where(v == 0, 0, t + 1)


def _body(x_vmem, o_hbm, obuf, osem):
    def out_copy(i, slot):
        return pltpu.make_async_copy(
            obuf.at[slot], o_hbm.at[pl.ds(i * _R, _R)], osem.at[slot])

    for i in range(_C):
        slot = i % _D
        if i >= _D:
            out_copy(i - _D, slot).wait()
        obuf[slot] = _bucket(x_vmem[pl.ds(i * _R, _R), :])
        out_copy(i, slot).start()
    for i in range(_C - _D, _C):
        out_copy(i, i % _D).wait()


def kernel(x):
    xt = x.T
    out_t = pl.pallas_call(
        _body,
        out_shape=jax.ShapeDtypeStruct((_ROWS, _COLS), jnp.int32),
        in_specs=[pl.BlockSpec(memory_space=pltpu.MemorySpace.VMEM)],
        out_specs=pl.BlockSpec(memory_space=pltpu.MemorySpace.HBM),
        scratch_shapes=[
            pltpu.VMEM((_D, _R, _COLS), jnp.int32),
            pltpu.SemaphoreType.DMA((_D,)),
        ],
    )(xt)
    return out_t.T
